# Initial kernel scaffold; baseline (speedup 1.0000x reference)
import jax
import jax.numpy as jnp
from jax.experimental import pallas as pl


def kernel(input1, input2):
    raise NotImplementedError("write your pallas kernel here")


# speedup vs baseline: 1.4646x; 1.4646x over previous
import jax
import jax.numpy as jnp
from jax.experimental import pallas as pl


def _chamfer_body(x_ref, yt_ref, dist_ref, idx_ref):
    x = x_ref[0]
    yt = yt_ref[0]
    nt = x.shape[0]
    m = yt.shape[1]
    d0 = x[:, 0:1] - yt[0:1, :]
    d1 = x[:, 1:2] - yt[1:2, :]
    d2 = x[:, 2:3] - yt[2:3, :]
    d = d0 * d0 + d1 * d1 + d2 * d2
    mn = jnp.min(d, axis=1, keepdims=True)
    iota = jax.lax.broadcasted_iota(jnp.int32, (nt, m), 1)
    idx = jnp.min(jnp.where(d == mn, iota, jnp.int32(m)), axis=1)
    dist_ref[0, 0, 0] = mn[:, 0]
    idx_ref[0, 0, 0] = idx


def kernel(input1, input2):
    b, n, _ = input1.shape
    m = input2.shape[1]
    nt = min(128, n)
    n_tiles = n // nt
    yt = jnp.transpose(input2, (0, 2, 1))
    dist, idx = pl.pallas_call(
        _chamfer_body,
        grid=(b, n_tiles),
        in_specs=[
            pl.BlockSpec((1, nt, 3), lambda bi, i: (bi, i, 0)),
            pl.BlockSpec((1, 3, m), lambda bi, i: (bi, 0, 0)),
        ],
        out_specs=[
            pl.BlockSpec((1, 1, 1, nt), lambda bi, i: (bi, i, 0, 0)),
            pl.BlockSpec((1, 1, 1, nt), lambda bi, i: (bi, i, 0, 0)),
        ],
        out_shape=[
            jax.ShapeDtypeStruct((b, n_tiles, 1, nt), jnp.float32),
            jax.ShapeDtypeStruct((b, n_tiles, 1, nt), jnp.int32),
        ],
    )(input1, yt)
    return dist.reshape(b, n), idx.reshape(b, n)
